# Initial kernel scaffold; baseline (speedup 1.0000x reference)
import jax
import jax.numpy as jnp
from jax.experimental import pallas as pl


def kernel(x, weight, bias):
    raise NotImplementedError("write your pallas kernel here")


# speedup vs baseline: 21.1934x; 21.1934x over previous
import functools

import jax
import jax.numpy as jnp
from jax.experimental import pallas as pl
from jax.experimental.pallas import tpu as pltpu

IMAGE_SIZE = 48
KERNEL_SIZE = 3
K = 256
OUT_DIM = IMAGE_SIZE - KERNEL_SIZE + 1
BATCH = 16


def _round_bf16(v):
    b = jax.lax.bitcast_convert_type(v, jnp.int32)
    lsb = jax.lax.shift_right_logical(b, 16) & 1
    b2 = (b + 0x7FFF + lsb) & ~0xFFFF
    return jax.lax.bitcast_convert_type(b2, jnp.float32)


def _trunc_conv_abs_kernel(w_ref, b_ref, x_ref, o_ref):
    L, OD, KS = IMAGE_SIZE, OUT_DIM, KERNEL_SIZE
    x = x_ref[...]
    B = x.shape[0]

    w = [[w_ref[ki * KS + kj] for kj in range(KS)] for ki in range(KS)]
    wr = [[_round_bf16(w_ref[ki * KS + kj]) for kj in range(KS)]
          for ki in range(KS)]
    xr = _round_bf16(x)

    c1 = jnp.zeros((B, OD, OD), jnp.float32)
    for ki in range(KS):
        for kj in range(KS):
            c1 = c1 + wr[ki][kj] * xr[:, ki:ki + OD, kj:kj + OD]
    ka = c1 * (1.0 / 9.0)

    zc = jnp.zeros((B, 2, OD), jnp.float32)
    p1 = jnp.concatenate([zc, ka, zc], axis=1)
    zr = jnp.zeros((B, L + 2, 2), jnp.float32)
    p = jnp.concatenate([zr, p1, zr], axis=2)
    bs = jnp.zeros((B, L, L), jnp.float32)
    for d1 in range(KS):
        for d2 in range(KS):
            bs = bs + p[:, d1:d1 + L, d2:d2 + L]

    ii = jax.lax.broadcasted_iota(jnp.int32, (L, L), 0)
    jj = jax.lax.broadcasted_iota(jnp.int32, (L, L), 1)
    ws = jnp.zeros((L, L), jnp.float32)
    for ki in range(KS):
        rmask = (ii >= ki) & (ii <= OD - 1 + ki)
        for kj in range(KS):
            cmask = (jj >= kj) & (jj <= OD - 1 + kj)
            ws = ws + jnp.where(rmask & cmask, w[ki][kj], 0.0)
    nrow = jnp.minimum(ii, 2) - jnp.maximum(ii - (OD - 1), 0) + 1
    ncol = jnp.minimum(jj, 2) - jnp.maximum(jj - (OD - 1), 0) + 1
    cnt = (nrow * ncol).astype(jnp.float32)

    r = cnt[None] * (x * ws[None] - bs)

    abits = jax.lax.bitcast_convert_type(jnp.abs(r), jnp.int32)

    def body(i, t):
        cand = t | jnp.left_shift(1, 30 - i)
        cnt_ge = jnp.sum((abits >= cand).astype(jnp.int32), axis=(1, 2),
                         keepdims=True)
        return jnp.where(cnt_ge >= K, cand, t)

    thr = jax.lax.fori_loop(0, 31, body, jnp.zeros((B, 1, 1), jnp.int32))

    xm = jnp.where(abits >= thr, 0.0, xr)

    out = jnp.zeros((B, OD, OD), jnp.float32)
    for ki in range(KS):
        for kj in range(KS):
            out = out + wr[ki][kj] * xm[:, ki:ki + OD, kj:kj + OD]
    o_ref[...] = out + b_ref[0]


@functools.partial(jax.jit, static_argnames=("interpret",))
def kernel(x, weight, bias, interpret=False):
    out = pl.pallas_call(
        _trunc_conv_abs_kernel,
        out_shape=jax.ShapeDtypeStruct((BATCH, OUT_DIM, OUT_DIM), jnp.float32),
        in_specs=[
            pl.BlockSpec(memory_space=pltpu.SMEM),
            pl.BlockSpec(memory_space=pltpu.SMEM),
            pl.BlockSpec(memory_space=pltpu.VMEM),
        ],
        out_specs=pl.BlockSpec(memory_space=pltpu.VMEM),
        interpret=interpret,
    )(weight.reshape(-1), bias.reshape(-1), x.reshape(BATCH, IMAGE_SIZE, IMAGE_SIZE))
    return out[:, None]
